# baseline (speedup 1.0000x reference)
import functools

import jax
import jax.numpy as jnp
from jax import lax
from jax.experimental import pallas as pl
from jax.experimental.pallas import tpu as pltpu
from jax.experimental.pallas import tpu_sc as plsc

_NROI = 32
_PH = 14
_PW = 14
_BINS = _PH * _PW
_PADB = 224
_NC = 2
_NS = 16
_NW = _NC * _NS
_TOT = _NROI * _PADB
_CHUNK = _TOT // _NW
_HALF = _CHUNK // 2
_LANES = 16


def _grid_body(m_ref, f_ref, o_ref, tab_ref, *, wm1, hm1, tabh, tabc):
    m00 = m_ref[:, 0:1]
    m01 = m_ref[:, 1:2]
    m02 = m_ref[:, 2:3]
    m10 = m_ref[:, 3:4]
    m11 = m_ref[:, 4:5]
    m12 = m_ref[:, 5:6]

    lane = lax.broadcasted_iota(jnp.int32, (_NROI, _PADB), 1)
    yi = lax.div(lane, _PW)
    xi = lane - yi * _PW
    x = xi.astype(jnp.float32)
    y = yi.astype(jnp.float32)
    xp = x + 1.0
    yp = y + 1.0

    p0 = m00 * x + m01 * y + m02
    p1 = m10 * x + m11 * y + m12
    p2 = m00 * x + m01 * yp + m02
    p3 = m10 * x + m11 * yp + m12
    p4 = m00 * xp + m01 * y + m02
    p5 = m10 * xp + m11 * y + m12
    p6 = m00 * xp + m01 * yp + m02
    p7 = m10 * xp + m11 * yp + m12

    left = jnp.maximum(jnp.round(jnp.minimum(jnp.minimum(p0, p2), jnp.minimum(p4, p6))), 0.0)
    right = jnp.minimum(jnp.round(jnp.maximum(jnp.maximum(p0, p2), jnp.maximum(p4, p6))), wm1)
    top = jnp.maximum(jnp.round(jnp.minimum(jnp.minimum(p1, p3), jnp.minimum(p5, p7))), 0.0)
    bottom = jnp.minimum(jnp.round(jnp.maximum(jnp.maximum(p1, p3), jnp.maximum(p5, p7))), hm1)

    bin_cx = (left + right) / 2.0
    bin_cy = (top + bottom) / 2.0
    fl_cx = jnp.floor(bin_cx)
    fl_cy = jnp.floor(bin_cy)
    rx = bin_cx - fl_cx
    ry = bin_cy - fl_cy

    ai_l = jnp.clip(fl_cx.astype(jnp.int32), 0, tabh - 1)
    ai_r = jnp.clip(jnp.ceil(bin_cx).astype(jnp.int32), 0, tabh - 1)
    bi_t = jnp.clip(fl_cy.astype(jnp.int32), 0, tabc - 1)
    bi_b = jnp.clip(jnp.ceil(bin_cy).astype(jnp.int32), 0, tabc - 1)

    o_ref[:, 0, :] = ai_l * tabc + bi_t
    o_ref[:, 1, :] = ai_r * tabc + bi_t
    o_ref[:, 2, :] = ai_r * tabc + bi_b
    o_ref[:, 3, :] = ai_l * tabc + bi_b
    o_ref[:, 4, :] = lax.bitcast_convert_type((1.0 - rx) * (1.0 - ry), jnp.int32)
    o_ref[:, 5, :] = lax.bitcast_convert_type(rx * (1.0 - ry), jnp.int32)
    o_ref[:, 6, :] = lax.bitcast_convert_type(rx * ry, jnp.int32)
    o_ref[:, 7, :] = lax.bitcast_convert_type((1.0 - rx) * ry, jnp.int32)

    tab_ref[...] = f_ref[0, 0]


def _grid_call(m, features, wm1, hm1, tabh, tabc):
    return pl.pallas_call(
        functools.partial(_grid_body, wm1=wm1, hm1=hm1, tabh=tabh, tabc=tabc),
        grid=(1,),
        in_specs=[
            pl.BlockSpec(m.shape, lambda i: (0, 0)),
            pl.BlockSpec((1, 1, tabh, tabc), lambda i: (0, 0, 0, 0)),
        ],
        out_specs=(
            pl.BlockSpec((_NROI, 8, _PADB), lambda i: (0, 0, 0)),
            pl.BlockSpec((tabh, tabc), lambda i: (0, 0)),
        ),
        out_shape=(
            jax.ShapeDtypeStruct((_NROI, 8, _PADB), jnp.int32),
            jax.ShapeDtypeStruct((tabh, tabc), jnp.float32),
        ),
    )(m, features)


@functools.cache
def _make_sc_gather(tabh, tabc, channel):
    mesh = plsc.VectorSubcoreMesh(
        core_axis_name="c", subcore_axis_name="s",
        num_cores=_NC, num_subcores=_NS)
    rows_per_sub = tabh // _NS

    @functools.partial(
        pl.kernel,
        out_type=(
            jax.ShapeDtypeStruct((_NROI, _BINS, channel), jnp.float32),
            jax.ShapeDtypeStruct((tabh * tabc,), jnp.float32),
        ),
        mesh=mesh,
        compiler_params=pltpu.CompilerParams(
            needs_layout_passes=False, use_tc_tiling_on_sc=False),
        scratch_types=[
            pltpu.VMEM((8, _CHUNK), jnp.int32),
            pltpu.VMEM((4, _CHUNK), jnp.float32),
            pltpu.VMEM((_CHUNK,), jnp.float32),
            pltpu.VMEM((_BINS, channel), jnp.float32),
            pltpu.VMEM((rows_per_sub, tabc), jnp.float32),
            pltpu.SemaphoreType.DMA,
            pltpu.SemaphoreType.DMA,
        ],
    )
    def sc_gather(tab_hbm, idxw_hbm, out_hbm, flat_hbm,
                  idxw_v, val_v, pool_v, rows_v, strip_v, sem, sem_idx):
        wid = lax.axis_index("s") * _NC + lax.axis_index("c")
        sub = lax.axis_index("s")

        cp_idx = pltpu.async_copy(idxw_hbm.at[wid], idxw_v, sem_idx)
        row_cps = []
        for k in range(rows_per_sub):
            r = sub * rows_per_sub + k
            row_cps.append(pltpu.async_copy(
                tab_hbm.at[r], flat_hbm.at[pl.ds(r * tabc, tabc)], sem))
        for cp in row_cps:
            cp.wait()
        plsc.subcore_barrier()

        cp_idx.wait()
        copies = []
        for c in range(4):
            for h in range(2):
                sl = pl.ds(h * _HALF, _HALF)
                copies.append(pltpu.async_copy(
                    flat_hbm.at[idxw_v.at[c, sl]], val_v.at[c, sl], sem))
        for cp in copies:
            cp.wait()
        for j in range(_CHUNK // _LANES):
            sl = pl.ds(j * _LANES, _LANES)
            acc = None
            for c in range(4):
                v = val_v[c, sl].astype(jnp.int32).astype(jnp.float32)
                w = plsc.bitcast(idxw_v[4 + c, sl], jnp.float32)
                acc = v * w if acc is None else acc + v * w
            pool_v[sl] = acc

        nvec = channel // _LANES

        def bcast_lanes(j, nlanes):
            vals = pool_v[pl.ds(j * _LANES, _LANES)]
            for l in range(nlanes):
                splat = jnp.full((_LANES,), vals[l], jnp.float32)
                b = j * _LANES + l
                for c in range(nvec):
                    rows_v[b, pl.ds(c * _LANES, _LANES)] = splat

        lax.fori_loop(0, _BINS // _LANES,
                      lambda j, carry: (bcast_lanes(j, _LANES), carry)[1], 0)
        bcast_lanes(_BINS // _LANES, _BINS % _LANES)
        pltpu.sync_copy(rows_v, out_hbm.at[wid])

    return sc_gather


def kernel(pooled_height, pooled_width, spatial_scale, features, rois):
    width = features.shape[1]
    height = features.shape[2]
    channel = features.shape[3]
    tabh = features.shape[2]
    tabc = features.shape[3]

    phf = jnp.asarray(pooled_height).astype(jnp.float32)
    pwf = jnp.asarray(pooled_width).astype(jnp.float32)
    ssf = jnp.asarray(spatial_scale).astype(jnp.float32)

    roi_idx = jnp.concatenate(
        [jnp.array([0], dtype=jnp.int32), jnp.arange(_NROI - 1, dtype=jnp.int32)])
    r = rois[0, roi_idx, :].astype(jnp.float32)
    a1, a2, a3, a4, a5 = r[:, 1], r[:, 2], r[:, 3], r[:, 4], r[:, 5]
    m5 = a5 * 180.0 * 3.1415926535
    roi_pw = (a4 / a3) * pwf
    dx = -roi_pw / 2.0
    dy = -phf / 2.0
    sx = (a4 / roi_pw) * ssf
    sy = a3 / (phf * ssf)
    alpha = jnp.cos(m5)
    beta = jnp.sin(m5)
    m00 = alpha * sx
    m01 = beta * sy
    m02 = m00 * dx + m01 * dy + a1 * ssf
    m10 = -beta * sx
    m11 = alpha * sy
    m12 = m10 * dx + m11 * dy + a2 * ssf
    m = jnp.stack([m00, m01, m02, m10, m11, m12], axis=1)

    idxw, tab = _grid_call(
        m, features, float(width - 1), float(height - 1), tabh, tabc)

    sc_gather = _make_sc_gather(tabh, tabc, channel)
    out3, _ = sc_gather(tab, idxw)
    return out3.reshape(_NROI, _PH, _PW, channel)


# speedup vs baseline: 1.3459x; 1.3459x over previous
import functools

import jax
import jax.numpy as jnp
from jax import lax
from jax.experimental import pallas as pl
from jax.experimental.pallas import tpu as pltpu
from jax.experimental.pallas import tpu_sc as plsc

_NROI = 32
_PH = 14
_PW = 14
_BINS = _PH * _PW
_PADB = 224
_NC = 2
_NS = 16
_NW = _NC * _NS
_TOT = _NROI * _PADB
_CHUNK = _TOT // _NW
_HALF = _CHUNK // 2
_LANES = 16


def _grid_body(m_ref, f_ref, o_ref, tab_ref, *, wm1, hm1, tabh, tabc):
    m00 = m_ref[:, 0:1]
    m01 = m_ref[:, 1:2]
    m02 = m_ref[:, 2:3]
    m10 = m_ref[:, 3:4]
    m11 = m_ref[:, 4:5]
    m12 = m_ref[:, 5:6]

    lane = lax.broadcasted_iota(jnp.int32, (_NROI, _PADB), 1)
    yi = lax.div(lane, _PW)
    xi = lane - yi * _PW
    x = xi.astype(jnp.float32)
    y = yi.astype(jnp.float32)
    xp = x + 1.0
    yp = y + 1.0

    p0 = m00 * x + m01 * y + m02
    p1 = m10 * x + m11 * y + m12
    p2 = m00 * x + m01 * yp + m02
    p3 = m10 * x + m11 * yp + m12
    p4 = m00 * xp + m01 * y + m02
    p5 = m10 * xp + m11 * y + m12
    p6 = m00 * xp + m01 * yp + m02
    p7 = m10 * xp + m11 * yp + m12

    left = jnp.maximum(jnp.round(jnp.minimum(jnp.minimum(p0, p2), jnp.minimum(p4, p6))), 0.0)
    right = jnp.minimum(jnp.round(jnp.maximum(jnp.maximum(p0, p2), jnp.maximum(p4, p6))), wm1)
    top = jnp.maximum(jnp.round(jnp.minimum(jnp.minimum(p1, p3), jnp.minimum(p5, p7))), 0.0)
    bottom = jnp.minimum(jnp.round(jnp.maximum(jnp.maximum(p1, p3), jnp.maximum(p5, p7))), hm1)

    bin_cx = (left + right) / 2.0
    bin_cy = (top + bottom) / 2.0
    fl_cx = jnp.floor(bin_cx)
    fl_cy = jnp.floor(bin_cy)
    rx = bin_cx - fl_cx
    ry = bin_cy - fl_cy

    ai_l = jnp.clip(fl_cx.astype(jnp.int32), 0, tabh - 1)
    ai_r = jnp.clip(jnp.ceil(bin_cx).astype(jnp.int32), 0, tabh - 1)
    bi_t = jnp.clip(fl_cy.astype(jnp.int32), 0, tabc - 1)
    bi_b = jnp.clip(jnp.ceil(bin_cy).astype(jnp.int32), 0, tabc - 1)

    o_ref[:, 0, :] = ai_l * tabc + bi_t
    o_ref[:, 1, :] = ai_r * tabc + bi_t
    o_ref[:, 2, :] = ai_r * tabc + bi_b
    o_ref[:, 3, :] = ai_l * tabc + bi_b
    o_ref[:, 4, :] = lax.bitcast_convert_type((1.0 - rx) * (1.0 - ry), jnp.int32)
    o_ref[:, 5, :] = lax.bitcast_convert_type(rx * (1.0 - ry), jnp.int32)
    o_ref[:, 6, :] = lax.bitcast_convert_type(rx * ry, jnp.int32)
    o_ref[:, 7, :] = lax.bitcast_convert_type((1.0 - rx) * ry, jnp.int32)

    tab_ref[...] = f_ref[0, 0]


def _grid_call(m, features, wm1, hm1, tabh, tabc):
    return pl.pallas_call(
        functools.partial(_grid_body, wm1=wm1, hm1=hm1, tabh=tabh, tabc=tabc),
        grid=(1,),
        in_specs=[
            pl.BlockSpec(m.shape, lambda i: (0, 0)),
            pl.BlockSpec((1, 1, tabh, tabc), lambda i: (0, 0, 0, 0)),
        ],
        out_specs=(
            pl.BlockSpec((_NROI, 8, _PADB), lambda i: (0, 0, 0)),
            pl.BlockSpec((tabh, tabc), lambda i: (0, 0)),
        ),
        out_shape=(
            jax.ShapeDtypeStruct((_NROI, 8, _PADB), jnp.int32),
            jax.ShapeDtypeStruct((tabh, tabc), jnp.float32),
        ),
    )(m, features)


@functools.cache
def _make_sc_gather(channel):
    mesh = plsc.VectorSubcoreMesh(
        core_axis_name="c", subcore_axis_name="s",
        num_cores=_NC, num_subcores=_NS)

    @functools.partial(
        pl.kernel,
        out_type=jax.ShapeDtypeStruct((_NROI, _BINS, channel), jnp.float32),
        mesh=mesh,
        compiler_params=pltpu.CompilerParams(
            needs_layout_passes=False, use_tc_tiling_on_sc=False),
        scratch_types=[
            pltpu.VMEM((8, _CHUNK), jnp.int32),
            pltpu.VMEM((4, _CHUNK), jnp.float32),
            pltpu.VMEM((_CHUNK,), jnp.float32),
            pltpu.VMEM((_BINS, channel), jnp.float32),
            pltpu.SemaphoreType.DMA,
        ],
    )
    def sc_gather(tab_hbm, idxw_hbm, out_hbm, idxw_v, val_v, pool_v, rows_v, sem):
        wid = lax.axis_index("s") * _NC + lax.axis_index("c")
        pltpu.sync_copy(idxw_hbm.at[wid], idxw_v)
        copies = []
        for c in range(4):
            for h in range(2):
                sl = pl.ds(h * _HALF, _HALF)
                copies.append(pltpu.async_copy(
                    tab_hbm.at[idxw_v.at[c, sl]], val_v.at[c, sl], sem))
        for cp in copies:
            cp.wait()
        for j in range(_CHUNK // _LANES):
            sl = pl.ds(j * _LANES, _LANES)
            acc = None
            for c in range(4):
                v = val_v[c, sl].astype(jnp.int32).astype(jnp.float32)
                w = plsc.bitcast(idxw_v[4 + c, sl], jnp.float32)
                acc = v * w if acc is None else acc + v * w
            pool_v[sl] = acc

        nvec = channel // _LANES

        def bcast_lanes(j, nlanes):
            vals = pool_v[pl.ds(j * _LANES, _LANES)]
            for l in range(nlanes):
                splat = jnp.full((_LANES,), vals[l], jnp.float32)
                b = j * _LANES + l
                for c in range(nvec):
                    rows_v[b, pl.ds(c * _LANES, _LANES)] = splat

        lax.fori_loop(0, _BINS // _LANES,
                      lambda j, carry: (bcast_lanes(j, _LANES), carry)[1], 0)
        bcast_lanes(_BINS // _LANES, _BINS % _LANES)
        pltpu.sync_copy(rows_v, out_hbm.at[wid])

    return sc_gather


def kernel(pooled_height, pooled_width, spatial_scale, features, rois):
    width = features.shape[1]
    height = features.shape[2]
    channel = features.shape[3]
    tabh = features.shape[2]
    tabc = features.shape[3]

    phf = jnp.asarray(pooled_height).astype(jnp.float32)
    pwf = jnp.asarray(pooled_width).astype(jnp.float32)
    ssf = jnp.asarray(spatial_scale).astype(jnp.float32)

    roi_idx = jnp.concatenate(
        [jnp.array([0], dtype=jnp.int32), jnp.arange(_NROI - 1, dtype=jnp.int32)])
    r = rois[0, roi_idx, :].astype(jnp.float32)
    a1, a2, a3, a4, a5 = r[:, 1], r[:, 2], r[:, 3], r[:, 4], r[:, 5]
    m5 = a5 * 180.0 * 3.1415926535
    roi_pw = (a4 / a3) * pwf
    dx = -roi_pw / 2.0
    dy = -phf / 2.0
    sx = (a4 / roi_pw) * ssf
    sy = a3 / (phf * ssf)
    alpha = jnp.cos(m5)
    beta = jnp.sin(m5)
    m00 = alpha * sx
    m01 = beta * sy
    m02 = m00 * dx + m01 * dy + a1 * ssf
    m10 = -beta * sx
    m11 = alpha * sy
    m12 = m10 * dx + m11 * dy + a2 * ssf
    m = jnp.stack([m00, m01, m02, m10, m11, m12], axis=1)

    idxw, tab = _grid_call(
        m, features, float(width - 1), float(height - 1), tabh, tabc)

    sc_gather = _make_sc_gather(channel)
    out3 = sc_gather(tab.reshape(-1), idxw)
    return out3.reshape(_NROI, _PH, _PW, channel)
